# Initial kernel scaffold; baseline (speedup 1.0000x reference)
import jax
import jax.numpy as jnp
from jax.experimental import pallas as pl


def kernel(x_num, x_cat, W0, W1, W2, W3, W4, W5, W6):
    raise NotImplementedError("write your pallas kernel here")


# speedup vs baseline: 2.7196x; 2.7196x over previous
import functools

import jax
import jax.numpy as jnp
from jax import lax
from jax.experimental import pallas as pl
from jax.experimental.pallas import tpu as pltpu
from jax.experimental.pallas import tpu_sc as plsc

_B = 16384
_NC = 2
_NS = 16
_NW = _NC * _NS
_RPW = _B // _NW
_CH = 128
_NCH = _RPW // _CH

_OUT_D = 322
_NF = 6


def _body(xn16, idx, T01, W2, W3, W4p, W5, W6, out,
          idx_v, g01, g2, g3, g4, g5, g6, xn, asm, gsem):
    wid = lax.axis_index("s") * _NC + lax.axis_index("c")
    base = wid * _RPW

    pltpu.sync_copy(idx.at[wid], idx_v)

    @pl.loop(0, _NCH)
    def _(k):
        rows = pl.ds(base + k * _CH, _CH)
        cps = [
            pltpu.async_copy(T01.at[idx_v.at[0 * _NCH + k]], g01, gsem),
            pltpu.async_copy(W2.at[idx_v.at[1 * _NCH + k]], g2, gsem),
            pltpu.async_copy(W3.at[idx_v.at[2 * _NCH + k]], g3, gsem),
            pltpu.async_copy(W4p.at[idx_v.at[3 * _NCH + k]], g4, gsem),
            pltpu.async_copy(W5.at[idx_v.at[4 * _NCH + k]], g5, gsem),
            pltpu.async_copy(W6.at[idx_v.at[5 * _NCH + k]], g6, gsem),
            pltpu.async_copy(xn16.at[rows, :], xn.at[:, pl.ds(8, 16)], gsem),
        ]
        for c in cps:
            c.wait()

        @pl.loop(0, _CH)
        def _(r):
            asm[r, pl.ds(0, 16)] = g01[r, :]
            v0, v1 = g2[r, pl.ds(0, 16)], g2[r, pl.ds(16, 16)]
            v2, v3 = g2[r, pl.ds(32, 16)], g2[r, pl.ds(34, 16)]
            asm[r, pl.ds(7, 16)] = v0
            asm[r, pl.ds(23, 16)] = v1
            asm[r, pl.ds(39, 16)] = v2
            asm[r, pl.ds(41, 16)] = v3
            v0, v1 = g3[r, pl.ds(0, 16)], g3[r, pl.ds(16, 16)]
            v2, v3 = g3[r, pl.ds(32, 16)], g3[r, pl.ds(34, 16)]
            asm[r, pl.ds(57, 16)] = v0
            asm[r, pl.ds(73, 16)] = v1
            asm[r, pl.ds(89, 16)] = v2
            asm[r, pl.ds(91, 16)] = v3
            asm[r, pl.ds(107, 16)] = g4[r, :]
            v0, v1 = g5[r, pl.ds(0, 16)], g5[r, pl.ds(16, 16)]
            v2, v3 = g5[r, pl.ds(32, 16)], g5[r, pl.ds(34, 16)]
            asm[r, pl.ds(109, 16)] = v0
            asm[r, pl.ds(125, 16)] = v1
            asm[r, pl.ds(141, 16)] = v2
            asm[r, pl.ds(143, 16)] = v3
            asm[r, pl.ds(209, 16)] = v0
            asm[r, pl.ds(225, 16)] = v1
            asm[r, pl.ds(241, 16)] = v2
            asm[r, pl.ds(243, 16)] = v3
            asm[r, pl.ds(306, 16)] = xn[r, pl.ds(5, 16)]
            v0, v1 = g6[r, pl.ds(0, 16)], g6[r, pl.ds(16, 16)]
            v2, v3 = g6[r, pl.ds(32, 16)], g6[r, pl.ds(34, 16)]
            asm[r, pl.ds(159, 16)] = v0
            asm[r, pl.ds(175, 16)] = v1
            asm[r, pl.ds(191, 16)] = v2
            asm[r, pl.ds(193, 16)] = v3
            asm[r, pl.ds(259, 16)] = v0
            asm[r, pl.ds(275, 16)] = v1
            asm[r, pl.ds(291, 16)] = v2
            asm[r, pl.ds(293, 16)] = v3

        pltpu.sync_copy(asm, out.at[rows, :])


_sc_embed = functools.partial(
    pl.kernel,
    out_type=jax.ShapeDtypeStruct((_B, _OUT_D), jnp.float32),
    mesh=plsc.VectorSubcoreMesh(core_axis_name="c", subcore_axis_name="s"),
    compiler_params=pltpu.CompilerParams(use_tc_tiling_on_sc=False),
    scratch_types=[
        pltpu.VMEM((_NF * _NCH, _CH), jnp.int32),
        pltpu.VMEM((_CH, 16), jnp.float32),
        pltpu.VMEM((_CH, 64), jnp.float32),
        pltpu.VMEM((_CH, 64), jnp.float32),
        pltpu.VMEM((_CH, 16), jnp.float32),
        pltpu.VMEM((_CH, 64), jnp.float32),
        pltpu.VMEM((_CH, 64), jnp.float32),
        pltpu.VMEM((_CH, 24), jnp.float32),
        pltpu.VMEM((_CH, _OUT_D), jnp.float32),
        pltpu.SemaphoreType.DMA,
    ],
)(_body)


def kernel(x_num, x_cat, W0, W1, W2, W3, W4, W5, W6):
    f32 = jnp.float32
    T01 = jnp.concatenate([
        jnp.repeat(W0.astype(f32), 8, axis=0),
        jnp.tile(W1.astype(f32), (5, 1)),
        jnp.zeros((40, 9), f32),
    ], axis=1)
    W4p = jnp.concatenate([W4.astype(f32), jnp.zeros((4, 14), f32)], axis=1)
    xn16 = jnp.concatenate([x_num.astype(f32), jnp.zeros((_B, 3), f32)], axis=1)
    def pad64(Wt):
        return jnp.concatenate(
            [Wt.astype(f32), jnp.zeros((Wt.shape[0], 14), f32)], axis=1)

    W2, W3, W5, W6 = pad64(W2), pad64(W3), pad64(W5), pad64(W6)

    xc = x_cat.astype(jnp.int32)
    cols = [xc[:, 0] * 8 + xc[:, 1], xc[:, 2], xc[:, 3], xc[:, 4], xc[:, 5],
            xc[:, 6]]
    xi = jnp.stack(cols).reshape(_NF, _NW, _NCH, _CH)
    idx = xi.transpose(1, 0, 2, 3).reshape(_NW, _NF * _NCH, _CH)
    return _sc_embed(xn16, idx, T01, W2, W3, W4p, W5, W6)
